# Initial kernel scaffold; baseline (speedup 1.0000x reference)
import jax
import jax.numpy as jnp
from jax.experimental import pallas as pl


def kernel(x, w1, w2):
    raise NotImplementedError("write your pallas kernel here")


# speedup vs baseline: 8.2006x; 8.2006x over previous
import jax
import jax.numpy as jnp
from jax import lax
from jax.experimental import pallas as pl
from jax.experimental.pallas import tpu as pltpu

B, N, D, C = 4, 1024, 4, 64
F = D * C
K = 20
OUT = 128
BR = 256
NB = N // BR


def _fused_body(x_ref, w1_ref, w2_ref, out_ref, dist_ref, wacc_ref):
    r = pl.program_id(1)
    x = x_ref[0]
    w1 = w1_ref[...]
    w1n = w1 * w1 / jnp.sum(w1 * w1)
    wmat = jnp.concatenate([w1n.T] * D, axis=1)

    sq = jnp.sum(x * x, axis=1, keepdims=True)
    row0 = pl.multiple_of(r * BR, BR)
    xb = x_ref[0, pl.ds(row0, BR), :]
    sqb = jnp.sum(xb * xb, axis=1, keepdims=True)
    dist = sqb - 2.0 * lax.dot_general(
        xb, x, (((1,), (1,)), ((), ())), preferred_element_type=jnp.float32
    ) + sq.T
    dist_ref[...] = dist
    wacc_ref[...] = jnp.zeros((BR, F), jnp.float32)

    iota = lax.broadcasted_iota(jnp.int32, (BR, N), 1)
    for k in range(K):
        d = dist_ref[...]
        m = jnp.min(d, axis=1, keepdims=True)
        cand = jnp.where(d <= m, iota, N)
        idx = jnp.min(cand, axis=1, keepdims=True)
        onehot = iota == idx
        sel = lax.dot_general(
            onehot.astype(jnp.float32), x, (((1,), (0,)), ((), ())),
            preferred_element_type=jnp.float32)
        wacc_ref[...] += sel * wmat[k : k + 1, :]
        dist_ref[...] = jnp.where(onehot, jnp.inf, d)

    w2 = w2_ref[...]
    w2n = w2 * w2 / jnp.sum(w2 * w2)
    wa = wacc_ref[...]
    for d in range(D):
        out_ref[0, :, d * OUT : (d + 1) * OUT] = lax.dot_general(
            wa[:, d * C : (d + 1) * C], w2n, (((1,), (1,)), ((), ())),
            preferred_element_type=jnp.float32)


def kernel(x, w1, w2):
    x_flat = x.reshape(B, N, F)
    out = pl.pallas_call(
        _fused_body,
        grid=(B, NB),
        in_specs=[
            pl.BlockSpec((1, N, F), lambda b, r: (b, 0, 0)),
            pl.BlockSpec((C, K), lambda b, r: (0, 0)),
            pl.BlockSpec((OUT, C), lambda b, r: (0, 0)),
        ],
        out_specs=pl.BlockSpec((1, BR, D * OUT), lambda b, r: (b, r, 0)),
        out_shape=jax.ShapeDtypeStruct((B, N, D * OUT), jnp.float32),
        scratch_shapes=[
            pltpu.VMEM((BR, N), jnp.float32),
            pltpu.VMEM((BR, F), jnp.float32),
        ],
    )(x_flat, w1, w2)
    return out.reshape(B, N, D, OUT)


# speedup vs baseline: 9.5330x; 1.1625x over previous
import jax
import jax.numpy as jnp
from jax import lax
from jax.experimental import pallas as pl
from jax.experimental.pallas import tpu as pltpu

B, N, D, C = 4, 1024, 4, 64
F = D * C
K = 20
OUT = 128
BR = 512
NB = N // BR


def _fused_body(x_ref, w1_ref, w2_ref, out_ref, dist_ref, wacc_ref):
    r = pl.program_id(1)
    x = x_ref[0]
    w1 = w1_ref[...]
    w1n = w1 * w1 / jnp.sum(w1 * w1)
    wmat = jnp.concatenate([w1n.T] * D, axis=1)

    sq = jnp.sum(x * x, axis=1, keepdims=True)
    row0 = pl.multiple_of(r * BR, BR)
    xb = x_ref[0, pl.ds(row0, BR), :]
    sqb = jnp.sum(xb * xb, axis=1, keepdims=True)
    dist = sqb - 2.0 * lax.dot_general(
        xb, x, (((1,), (1,)), ((), ())), preferred_element_type=jnp.float32
    ) + sq.T
    dist_ref[...] = dist
    wacc_ref[...] = jnp.zeros((BR, F), jnp.float32)

    iota_f = lax.broadcasted_iota(jnp.int32, (BR, N), 1).astype(jnp.float32)
    for k in range(K):
        d = dist_ref[...]
        m = jnp.min(d, axis=1, keepdims=True)
        cand = jnp.where(d <= m, iota_f, float(N))
        idxf = jnp.min(cand, axis=1, keepdims=True)
        msk = cand == idxf
        onehot = jnp.where(msk, 1.0, 0.0)
        sel = lax.dot_general(
            onehot, x, (((1,), (0,)), ((), ())),
            preferred_element_type=jnp.float32)
        wacc_ref[...] += sel * wmat[k : k + 1, :]
        dist_ref[...] = jnp.where(msk, 3.0e38, d)

    w2 = w2_ref[...]
    w2n = w2 * w2 / jnp.sum(w2 * w2)
    wa = wacc_ref[...]
    for d in range(D):
        out_ref[0, :, d * OUT : (d + 1) * OUT] = lax.dot_general(
            wa[:, d * C : (d + 1) * C], w2n, (((1,), (1,)), ((), ())),
            preferred_element_type=jnp.float32)


def kernel(x, w1, w2):
    x_flat = x.reshape(B, N, F)
    out = pl.pallas_call(
        _fused_body,
        grid=(B, NB),
        in_specs=[
            pl.BlockSpec((1, N, F), lambda b, r: (b, 0, 0)),
            pl.BlockSpec((C, K), lambda b, r: (0, 0)),
            pl.BlockSpec((OUT, C), lambda b, r: (0, 0)),
        ],
        out_specs=pl.BlockSpec((1, BR, D * OUT), lambda b, r: (b, r, 0)),
        out_shape=jax.ShapeDtypeStruct((B, N, D * OUT), jnp.float32),
        scratch_shapes=[
            pltpu.VMEM((BR, N), jnp.float32),
            pltpu.VMEM((BR, F), jnp.float32),
        ],
    )(x_flat, w1, w2)
    return out.reshape(B, N, D, OUT)
